# baseline (speedup 1.0000x reference)
import functools

import jax
import jax.numpy as jnp
from jax import lax
from jax.experimental import pallas as pl
from jax.experimental.pallas import tpu as pltpu
from jax.experimental.pallas import tpu_sc as plsc

_N = 5000
_NPAD = 5120
_NW = 32
_ROWS = _NPAD // _NW
_NV = _NPAD // 16
_NUM_PAD = _NPAD - _N


def _lane_bcast(vec, lane_splat):
    return lax.gather(
        vec,
        lane_splat[:, None],
        lax.GatherDimensionNumbers(
            offset_dims=(), collapsed_slice_dims=(0,), start_index_map=(0,)
        ),
        slice_sizes=(1,),
        mode=lax.GatherScatterMode.PROMISE_IN_BOUNDS,
    )


def _sc_body(comp_hbm, out_hbm, comp_v, key_v, b_v, a_v, stage_v):
    cid = lax.axis_index("c")
    sid = lax.axis_index("s")
    wid = sid * 2 + cid

    pltpu.sync_copy(comp_hbm, comp_v)

    iota = lax.iota(jnp.int32, 16)
    rots = [(iota + s) & 15 for s in (8, 4, 2, 1)]

    def allsum(v):
        for r in rots:
            v = v + _lane_bcast(v, r)
        return v

    def build(j, _):
        sl = pl.ds(j * 16, 16)
        p_l = comp_v[0, sl]
        p_t = comp_v[1, sl]
        p_r = comp_v[2, sl]
        p_b = comp_v[3, sl]
        t_l = comp_v[4, sl]
        t_t = comp_v[5, sl]
        t_r = comp_v[6, sl]
        t_b = comp_v[7, sl]
        cen = comp_v[8, sl]
        target_area = (t_l + t_r) * (t_t + t_b)
        pred_area = (p_l + p_r) * (p_t + p_b)
        w_int = jnp.minimum(p_l, t_l) + jnp.minimum(p_r, t_r)
        h_int = jnp.minimum(p_b, t_b) + jnp.minimum(p_t, t_t)
        area_int = w_int * h_int
        area_union = target_area + pred_area - area_int
        iou = (area_int + 1.0) / (area_union + 1.0)
        valid = (iota + j * 16) < _N
        key_v[sl] = jnp.where(valid, iou, -1.0)
        b_v[sl] = jnp.where(valid, jnp.exp(-cen), 0.0)
        a_v[sl] = jnp.exp(-3.0 * cen)
        return 0

    lax.fori_loop(0, _NV, build, 0, unroll=2)

    row0 = wid * _ROWS
    zf = jnp.zeros((16,), jnp.float32)
    zi = jnp.zeros((16,), jnp.int32)

    def row_body(r, acc):
        k = row0 + r
        k_splat = jnp.full((16,), k, jnp.int32)
        kbase = (k // 16) * 16
        lane_splat = k_splat - kbase
        kf = _lane_bcast(key_v[pl.ds(kbase, 16)], lane_splat)
        af = _lane_bcast(a_v[pl.ds(kbase, 16)], lane_splat)

        def lstep(j, carry):
            t_acc, n_acc = carry
            sl = pl.ds(j * 16, 16)
            kl = key_v[sl]
            bl = b_v[sl]
            c_lt = kl < kf
            c_eq = kl == kf
            c_ix = iota > (k - j * 16)
            below = c_lt | (c_eq & c_ix)
            t_acc = t_acc + jnp.where(below, bl, 0.0)
            n_acc = n_acc + jnp.where(below, 1, 0)
            return (t_acc, n_acc)

        t_vec, n_vec = lax.fori_loop(0, _NV, lstep, (zf, zi), unroll=2)
        t_tot = allsum(t_vec)
        cnt = allsum(n_vec) - _NUM_PAD
        valid_i = jnp.where(cnt > 0, 1, 0)
        row_i = jnp.where(k_splat < _N, 1, 0)
        lane_i = jnp.where(iota == 0, 1, 0)
        ok_i = valid_i * row_i * lane_i
        cntf = jnp.where(cnt > 0, cnt, 1).astype(jnp.float32)
        contrib = jnp.where(ok_i > 0, af * t_tot / cntf, 0.0)
        return acc + contrib

    acc = lax.fori_loop(0, _ROWS, row_body, zf)
    stage_v[...] = acc
    pltpu.sync_copy(stage_v, out_hbm.at[wid])


@jax.jit
def _run(comp):
    f = functools.partial(
        pl.kernel,
        mesh=plsc.VectorSubcoreMesh(core_axis_name="c", subcore_axis_name="s"),
        out_type=jax.ShapeDtypeStruct((_NW, 16), jnp.float32),
        scratch_types=[
            pltpu.VMEM((9, _NPAD), jnp.float32),
            pltpu.VMEM((_NPAD,), jnp.float32),
            pltpu.VMEM((_NPAD,), jnp.float32),
            pltpu.VMEM((_NPAD,), jnp.float32),
            pltpu.VMEM((16,), jnp.float32),
        ],
    )(_sc_body)
    return f(comp)


def kernel(centerness_flatten, centerness_targets, box_regression_flatten, reg_targets_flatten):
    comp = jnp.zeros((9, _NPAD), jnp.float32)
    comp = comp.at[0:4, :_N].set(reg_targets_flatten.T)
    comp = comp.at[4:8, :_N].set(box_regression_flatten.T)
    comp = comp.at[8, :_N].set(centerness_flatten)
    partials = _run(comp)
    return jnp.sum(partials) / jnp.float32(_N - 1)


# speedup vs baseline: 1.0077x; 1.0077x over previous
import functools

import jax
import jax.numpy as jnp
from jax import lax
from jax.experimental import pallas as pl
from jax.experimental.pallas import tpu as pltpu
from jax.experimental.pallas import tpu_sc as plsc

_N = 5000
_NPAD = 5120
_NW = 32
_ROWS = _NPAD // _NW
_NV = _NPAD // 16
_NUM_PAD = _NPAD - _N


def _lane_bcast(vec, lane_splat):
    return lax.gather(
        vec,
        lane_splat[:, None],
        lax.GatherDimensionNumbers(
            offset_dims=(), collapsed_slice_dims=(0,), start_index_map=(0,)
        ),
        slice_sizes=(1,),
        mode=lax.GatherScatterMode.PROMISE_IN_BOUNDS,
    )


def _sc_body(comp_hbm, out_hbm, comp_v, key_v, b_v, a_v, stage_v):
    cid = lax.axis_index("c")
    sid = lax.axis_index("s")
    wid = sid * 2 + cid

    pltpu.sync_copy(comp_hbm, comp_v)

    iota = lax.iota(jnp.int32, 16)
    rots = [(iota + s) & 15 for s in (8, 4, 2, 1)]

    def allsum(v):
        for r in rots:
            v = v + _lane_bcast(v, r)
        return v

    def build(j, _):
        sl = pl.ds(j * 16, 16)
        p_l = comp_v[0, sl]
        p_t = comp_v[1, sl]
        p_r = comp_v[2, sl]
        p_b = comp_v[3, sl]
        t_l = comp_v[4, sl]
        t_t = comp_v[5, sl]
        t_r = comp_v[6, sl]
        t_b = comp_v[7, sl]
        cen = comp_v[8, sl]
        target_area = (t_l + t_r) * (t_t + t_b)
        pred_area = (p_l + p_r) * (p_t + p_b)
        w_int = jnp.minimum(p_l, t_l) + jnp.minimum(p_r, t_r)
        h_int = jnp.minimum(p_b, t_b) + jnp.minimum(p_t, t_t)
        area_int = w_int * h_int
        area_union = target_area + pred_area - area_int
        iou = (area_int + 1.0) / (area_union + 1.0)
        valid = (iota + j * 16) < _N
        key_v[sl] = jnp.where(valid, iou, -1.0)
        b_v[sl] = jnp.where(valid, jnp.exp(-cen), 0.0)
        a_v[sl] = jnp.exp(-3.0 * cen)
        return 0

    lax.fori_loop(0, _NV, build, 0, unroll=2)

    row0 = wid * _ROWS
    zf = jnp.zeros((16,), jnp.float32)
    zi = jnp.zeros((16,), jnp.int32)

    def row_body(r, acc):
        k = row0 + r
        k_splat = jnp.full((16,), k, jnp.int32)
        kf = plsc.load_gather(key_v, [k_splat])
        af = plsc.load_gather(a_v, [k_splat])

        def lstep(j, carry):
            t_acc, n_acc = carry
            sl = pl.ds(j * 16, 16)
            kl = key_v[sl]
            bl = b_v[sl]
            c_lt = kl < kf
            c_eq = kl == kf
            c_ix = iota > (k - j * 16)
            below = c_lt | (c_eq & c_ix)
            t_acc = t_acc + jnp.where(below, bl, 0.0)
            n_acc = n_acc + plsc.all_reduce_population_count(below)
            return (t_acc, n_acc)

        t_vec, n_vec = lax.fori_loop(0, _NV, lstep, (zf, zi), unroll=2)
        t_tot = allsum(t_vec)
        cnt = n_vec - _NUM_PAD
        valid_i = jnp.where(cnt > 0, 1, 0)
        row_i = jnp.where(k_splat < _N, 1, 0)
        lane_i = jnp.where(iota == 0, 1, 0)
        ok_i = valid_i * row_i * lane_i
        cntf = jnp.where(cnt > 0, cnt, 1).astype(jnp.float32)
        contrib = jnp.where(ok_i > 0, af * t_tot / cntf, 0.0)
        return acc + contrib

    acc = lax.fori_loop(0, _ROWS, row_body, zf)
    stage_v[...] = acc
    pltpu.sync_copy(stage_v, out_hbm.at[wid])


@jax.jit
def _run(comp):
    f = functools.partial(
        pl.kernel,
        mesh=plsc.VectorSubcoreMesh(core_axis_name="c", subcore_axis_name="s"),
        compiler_params=pltpu.CompilerParams(needs_layout_passes=False),
        out_type=jax.ShapeDtypeStruct((_NW, 16), jnp.float32),
        scratch_types=[
            pltpu.VMEM((9, _NPAD), jnp.float32),
            pltpu.VMEM((_NPAD,), jnp.float32),
            pltpu.VMEM((_NPAD,), jnp.float32),
            pltpu.VMEM((_NPAD,), jnp.float32),
            pltpu.VMEM((16,), jnp.float32),
        ],
    )(_sc_body)
    return f(comp)


def kernel(centerness_flatten, centerness_targets, box_regression_flatten, reg_targets_flatten):
    comp = jnp.zeros((9, _NPAD), jnp.float32)
    comp = comp.at[0:4, :_N].set(reg_targets_flatten.T)
    comp = comp.at[4:8, :_N].set(box_regression_flatten.T)
    comp = comp.at[8, :_N].set(centerness_flatten)
    partials = _run(comp)
    return jnp.sum(partials) / jnp.float32(_N - 1)


# speedup vs baseline: 2.0105x; 1.9951x over previous
import functools

import jax
import jax.numpy as jnp
from jax import lax
from jax.experimental import pallas as pl
from jax.experimental.pallas import tpu as pltpu
from jax.experimental.pallas import tpu_sc as plsc

_N = 5000
_NW = 32
_CH = 160
_NPAD = _NW * _CH
_CV = _CH // 16
_CPAD = 256
_NUM_PAD = _NPAD - _N
_IMAX = 2147483647


def _iou_vecs(own_v, vj, iota, base):
    def comp(i):
        return own_v[pl.ds(i * _CH + vj * 16, 16)]
    p_l = comp(0)
    p_t = comp(1)
    p_r = comp(2)
    p_b = comp(3)
    t_l = comp(4)
    t_t = comp(5)
    t_r = comp(6)
    t_b = comp(7)
    cen = comp(8)
    target_area = (t_l + t_r) * (t_t + t_b)
    pred_area = (p_l + p_r) * (p_t + p_b)
    w_int = jnp.minimum(p_l, t_l) + jnp.minimum(p_r, t_r)
    h_int = jnp.minimum(p_b, t_b) + jnp.minimum(p_t, t_t)
    area_int = w_int * h_int
    area_union = target_area + pred_area - area_int
    iou = (area_int + 1.0) / (area_union + 1.0)
    gidx = iota + (base + vj * 16)
    valid = gidx < _N
    ikey = plsc.bitcast(jnp.where(valid, iou, 0.0), jnp.int32)
    return ikey, gidx, valid, cen


def _copy_own_rows(comp_hbm, own_v, base):
    for i in range(9):
        pltpu.sync_copy(
            comp_hbm.at[pl.ds(i * _NPAD + base, _CH)],
            own_v.at[pl.ds(i * _CH, _CH)],
        )


def _k1_body(comp_hbm, key_hbm, idx_hbm, pre_hbm,
             own_v, ikey_v, gidx_v, b_v, skey_v, sidx_v, sb_v, spre_v):
    cid = lax.axis_index("c")
    sid = lax.axis_index("s")
    wid = sid * 2 + cid
    base = wid * _CH
    iota = lax.iota(jnp.int32, 16)
    lane15 = jnp.full((16,), 15, jnp.int32)

    _copy_own_rows(comp_hbm, own_v, base)

    for vj in range(_CV):
        sl = pl.ds(vj * 16, 16)
        ikey, gidx, valid, cen = _iou_vecs(own_v, vj, iota, base)
        ikey_v[sl] = ikey
        gidx_v[sl] = gidx
        b_v[sl] = jnp.where(valid, jnp.exp(-cen), 0.0)

    for vj in range(_CV, _CPAD // 16):
        sl = pl.ds(vj * 16, 16)
        skey_v[sl] = jnp.full((16,), _IMAX, jnp.int32)
        sidx_v[sl] = jnp.full((16,), -1, jnp.int32)
        sb_v[sl] = jnp.zeros((16,), jnp.float32)

    rots = [(iota + r) & 15 for r in range(16)]

    def _rot(v, r):
        return lax.gather(
            v, rots[r][:, None],
            lax.GatherDimensionNumbers(
                offset_dims=(), collapsed_slice_dims=(0,), start_index_map=(0,)
            ),
            slice_sizes=(1,),
            mode=lax.GatherScatterMode.PROMISE_IN_BOUNDS,
        )

    def rank_rv(rv, _):
        slr = pl.ds(rv * 16, 16)
        kr = ikey_v[slr]
        gr = gidx_v[slr]

        def rank_cv(cv, n_acc):
            slc = pl.ds(cv * 16, 16)
            kc = ikey_v[slc]
            gc = gidx_v[slc]
            for r in range(16):
                kx = _rot(kc, r)
                gx = _rot(gc, r)
                below = (kx < kr) | ((kx == kr) & (gx > gr))
                n_acc = n_acc + jnp.where(below, 1, 0)
            return n_acc

        rank = lax.fori_loop(0, _CV, rank_cv, jnp.zeros((16,), jnp.int32))
        plsc.store_scatter(skey_v, [rank], kr)
        plsc.store_scatter(sidx_v, [rank], gr)
        plsc.store_scatter(sb_v, [rank], b_v[slr])
        return 0

    lax.fori_loop(0, _CV, rank_rv, 0)

    carry = jnp.zeros((16,), jnp.float32)
    for vj in range(_CV):
        sl = pl.ds(vj * 16, 16)
        bv = sb_v[sl]
        inc = plsc.cumsum(bv)
        spre_v[sl] = carry + (inc - bv)
        carry = carry + lax.gather(
            inc, lane15[:, None],
            lax.GatherDimensionNumbers(
                offset_dims=(), collapsed_slice_dims=(0,), start_index_map=(0,)
            ),
            slice_sizes=(1,),
            mode=lax.GatherScatterMode.PROMISE_IN_BOUNDS,
        )
    spre_v[pl.ds(_CH, 16)] = carry

    pltpu.sync_copy(skey_v, key_hbm.at[pl.ds(wid * _CPAD, _CPAD)])
    pltpu.sync_copy(sidx_v, idx_hbm.at[pl.ds(wid * _CPAD, _CPAD)])
    pltpu.sync_copy(spre_v, pre_hbm.at[pl.ds(wid * _CPAD, _CPAD)])


def _k2_body(comp_hbm, key_hbm, idx_hbm, pre_hbm, out_hbm,
             own_v, keyf_v, idxf_v, pref_v, stage_v):
    cid = lax.axis_index("c")
    sid = lax.axis_index("s")
    wid = sid * 2 + cid
    base = wid * _CH
    iota = lax.iota(jnp.int32, 16)

    pltpu.sync_copy(key_hbm, keyf_v)
    pltpu.sync_copy(idx_hbm, idxf_v)
    pltpu.sync_copy(pre_hbm, pref_v)
    _copy_own_rows(comp_hbm, own_v, base)

    def row_vreg(rv, acc):
        ikey, gidx, valid, cen = _iou_vecs(own_v, rv, iota, base)
        av = jnp.exp(-3.0 * cen)

        def group(g, carry):
            t_acc, n_acc = carry
            pos = [jnp.zeros((16,), jnp.int32) for _ in range(8)]
            for step in (128, 64, 32, 16, 8, 4, 2, 1):
                for cc in range(8):
                    cbase = (g * 8 + cc) * _CPAD
                    cand = pos[cc] + step
                    probe = cand + (cbase - 1)
                    pk = plsc.load_gather(keyf_v, [probe])
                    pi = plsc.load_gather(idxf_v, [probe])
                    below = (pk < ikey) | ((pk == ikey) & (pi > gidx))
                    pos[cc] = jnp.where(below, cand, pos[cc])
            for cc in range(8):
                cbase = (g * 8 + cc) * _CPAD
                n_acc = n_acc + pos[cc]
                t_acc = t_acc + plsc.load_gather(pref_v, [pos[cc] + cbase])
            return (t_acc, n_acc)

        t_vec, n_vec = lax.fori_loop(
            0, _NW // 8, group,
            (jnp.zeros((16,), jnp.float32), jnp.zeros((16,), jnp.int32)),
        )
        cnt = n_vec - _NUM_PAD
        ok_i = jnp.where(cnt > 0, 1, 0) * jnp.where(valid, 1, 0)
        cntf = jnp.where(cnt > 0, cnt, 1).astype(jnp.float32)
        return acc + jnp.where(ok_i > 0, av * t_vec / cntf, 0.0)

    acc = lax.fori_loop(0, _CV, row_vreg, jnp.zeros((16,), jnp.float32))
    stage_v[...] = acc
    pltpu.sync_copy(stage_v, out_hbm.at[wid])


_PUB = _NW * _CPAD


@jax.jit
def _run(comp):
    mesh = plsc.VectorSubcoreMesh(core_axis_name="c", subcore_axis_name="s")
    params = pltpu.CompilerParams(needs_layout_passes=False)

    k1 = functools.partial(
        pl.kernel, mesh=mesh, compiler_params=params,
        out_type=(
            jax.ShapeDtypeStruct((_PUB,), jnp.int32),
            jax.ShapeDtypeStruct((_PUB,), jnp.int32),
            jax.ShapeDtypeStruct((_PUB,), jnp.float32),
        ),
        scratch_types=[
            pltpu.VMEM((9 * _CH,), jnp.float32),
            pltpu.VMEM((_CH,), jnp.int32),
            pltpu.VMEM((_CH,), jnp.int32),
            pltpu.VMEM((_CH,), jnp.float32),
            pltpu.VMEM((_CPAD,), jnp.int32),
            pltpu.VMEM((_CPAD,), jnp.int32),
            pltpu.VMEM((_CPAD,), jnp.float32),
            pltpu.VMEM((_CPAD,), jnp.float32),
        ],
    )(_k1_body)
    key_p, idx_p, pre_p = k1(comp)

    k2 = functools.partial(
        pl.kernel, mesh=mesh, compiler_params=params,
        out_type=jax.ShapeDtypeStruct((_NW, 16), jnp.float32),
        scratch_types=[
            pltpu.VMEM((9 * _CH,), jnp.float32),
            pltpu.VMEM((_PUB,), jnp.int32),
            pltpu.VMEM((_PUB,), jnp.int32),
            pltpu.VMEM((_PUB,), jnp.float32),
            pltpu.VMEM((16,), jnp.float32),
        ],
    )(_k2_body)
    return k2(comp, key_p, idx_p, pre_p)


def kernel(centerness_flatten, centerness_targets, box_regression_flatten, reg_targets_flatten):
    comp = jnp.zeros((9, _NPAD), jnp.float32)
    comp = comp.at[0:4, :_N].set(reg_targets_flatten.T)
    comp = comp.at[4:8, :_N].set(box_regression_flatten.T)
    comp = comp.at[8, :_N].set(centerness_flatten)
    comp = comp.reshape(9 * _NPAD)
    partials = _run(comp)
    return jnp.sum(partials) / jnp.float32(_N - 1)


# speedup vs baseline: 2.2982x; 1.1431x over previous
import functools

import jax
import jax.numpy as jnp
from jax import lax
from jax.experimental import pallas as pl
from jax.experimental.pallas import tpu as pltpu
from jax.experimental.pallas import tpu_sc as plsc

_N = 5000
_NW = 32
_CH = 160
_NPAD = _NW * _CH
_CV = _CH // 16
_CPAD = 256
_NUM_PAD = _NPAD - _N
_IMAX = 2147483647
_GD = lax.GatherDimensionNumbers(
    offset_dims=(), collapsed_slice_dims=(0,), start_index_map=(0,)
)


def _perm(v, idx):
    return lax.gather(
        v, idx[:, None], _GD, slice_sizes=(1,),
        mode=lax.GatherScatterMode.PROMISE_IN_BOUNDS,
    )


def _iou_vecs(own_v, vj, iota, base):
    def comp(i):
        return own_v[pl.ds(i * _CH + vj * 16, 16)]
    p_l = comp(0)
    p_t = comp(1)
    p_r = comp(2)
    p_b = comp(3)
    t_l = comp(4)
    t_t = comp(5)
    t_r = comp(6)
    t_b = comp(7)
    cen = comp(8)
    target_area = (t_l + t_r) * (t_t + t_b)
    pred_area = (p_l + p_r) * (p_t + p_b)
    w_int = jnp.minimum(p_l, t_l) + jnp.minimum(p_r, t_r)
    h_int = jnp.minimum(p_b, t_b) + jnp.minimum(p_t, t_t)
    area_int = w_int * h_int
    area_union = target_area + pred_area - area_int
    iou = (area_int + 1.0) / (area_union + 1.0)
    gidx = iota + (base + vj * 16)
    valid = gidx < _N
    ikey = plsc.bitcast(jnp.where(valid, iou, 0.0), jnp.int32)
    return ikey, valid, cen


def _copy_own_rows(comp_hbm, own_v, base):
    for i in range(9):
        pltpu.sync_copy(
            comp_hbm.at[pl.ds(i * _NPAD + base, _CH)],
            own_v.at[pl.ds(i * _CH, _CH)],
        )


def _k1_body(comp_hbm, key_hbm, pre_hbm, rnk_hbm,
             own_v, ikey_v, b_v, rank_v, skey_v, sb_v, spre_v):
    cid = lax.axis_index("c")
    sid = lax.axis_index("s")
    wid = sid * 2 + cid
    base = wid * _CH
    iota = lax.iota(jnp.int32, 16)
    lane15 = jnp.full((16,), 15, jnp.int32)

    _copy_own_rows(comp_hbm, own_v, base)

    for vj in range(_CV):
        sl = pl.ds(vj * 16, 16)
        ikey, valid, cen = _iou_vecs(own_v, vj, iota, base)
        ikey_v[sl] = ikey
        b_v[sl] = jnp.where(valid, jnp.exp(-cen), 0.0)

    for vj in range(_CV, _CPAD // 16):
        sl = pl.ds(vj * 16, 16)
        skey_v[sl] = jnp.full((16,), _IMAX, jnp.int32)

    rots = [(iota + r) & 15 for r in range(16)]

    def rank_rv(rv, _):
        slr = pl.ds(rv * 16, 16)
        kr = ikey_v[slr]
        lr = iota + rv * 16

        def rank_cv(cv, n_acc):
            kc = ikey_v[pl.ds(cv * 16, 16)]
            for r in range(16):
                kx = _perm(kc, rots[r])
                lx = rots[r] + cv * 16
                below = (kx < kr) | ((kx == kr) & (lx > lr))
                n_acc = n_acc + jnp.where(below, 1, 0)
            return n_acc

        rank = lax.fori_loop(0, _CV, rank_cv, jnp.zeros((16,), jnp.int32))
        rank_v[slr] = rank
        plsc.store_scatter(skey_v, [rank], kr)
        plsc.store_scatter(sb_v, [rank], b_v[slr])
        return 0

    lax.fori_loop(0, _CV, rank_rv, 0)

    carry = jnp.zeros((16,), jnp.float32)
    for vj in range(_CV):
        sl = pl.ds(vj * 16, 16)
        bv = sb_v[sl]
        inc = plsc.cumsum(bv)
        spre_v[sl] = carry + (inc - bv)
        carry = carry + _perm(inc, lane15)
    spre_v[pl.ds(_CH, 16)] = carry

    pltpu.sync_copy(skey_v, key_hbm.at[pl.ds(wid * _CPAD, _CPAD)])
    pltpu.sync_copy(spre_v, pre_hbm.at[pl.ds(wid * _CPAD, _CPAD)])
    pltpu.sync_copy(rank_v, rnk_hbm.at[pl.ds(wid * _CH, _CH)])


def _k2_body(comp_hbm, key_hbm, pre_hbm, rnk_hbm, out_hbm,
             own_v, keyf_v, pref_v, rank_v, stage_v):
    cid = lax.axis_index("c")
    sid = lax.axis_index("s")
    wid = sid * 2 + cid
    base = wid * _CH
    iota = lax.iota(jnp.int32, 16)

    pltpu.sync_copy(key_hbm, keyf_v)
    pltpu.sync_copy(pre_hbm, pref_v)
    pltpu.sync_copy(rnk_hbm.at[pl.ds(base, _CH)], rank_v)
    _copy_own_rows(comp_hbm, own_v, base)

    def row_vreg(rv, acc):
        ikey, valid, cen = _iou_vecs(own_v, rv, iota, base)
        av = jnp.exp(-3.0 * cen)

        pos_own = rank_v[pl.ds(rv * 16, 16)]
        t0 = plsc.load_gather(pref_v, [pos_own + wid * _CPAD])

        def group(g, carry):
            t_acc, n_acc = carry
            cbase = g * 16
            pos = [jnp.zeros((16,), jnp.int32) for _ in range(16)]
            for step in (128, 64, 32, 16, 8, 4, 2, 1):
                for cc in range(16):
                    c = cbase + cc
                    after = jnp.full((16,), c, jnp.int32) > wid
                    cand = pos[cc] + step
                    pk = plsc.load_gather(keyf_v, [cand + (c * _CPAD - 1)])
                    below = jnp.where(after, pk <= ikey, pk < ikey)
                    pos[cc] = jnp.where(below, cand, pos[cc])
            for cc in range(16):
                c = cbase + cc
                skip = jnp.full((16,), c, jnp.int32) == wid
                p = jnp.where(skip, 0, pos[cc])
                n_acc = n_acc + p
                t_acc = t_acc + plsc.load_gather(pref_v, [p + c * _CPAD])
            return (t_acc, n_acc)

        t_vec, n_vec = lax.fori_loop(
            0, 2, group, (t0, pos_own))
        cnt = n_vec - _NUM_PAD
        ok_i = jnp.where(cnt > 0, 1, 0) * jnp.where(valid, 1, 0)
        cntf = jnp.where(cnt > 0, cnt, 1).astype(jnp.float32)
        return acc + jnp.where(ok_i > 0, av * t_vec / cntf, 0.0)

    acc = lax.fori_loop(0, _CV, row_vreg, jnp.zeros((16,), jnp.float32))
    stage_v[...] = acc
    pltpu.sync_copy(stage_v, out_hbm.at[wid])


_PUB = _NW * _CPAD


@jax.jit
def _run(comp):
    mesh = plsc.VectorSubcoreMesh(core_axis_name="c", subcore_axis_name="s")
    params = pltpu.CompilerParams(needs_layout_passes=False)

    k1 = functools.partial(
        pl.kernel, mesh=mesh, compiler_params=params,
        out_type=(
            jax.ShapeDtypeStruct((_PUB,), jnp.int32),
            jax.ShapeDtypeStruct((_PUB,), jnp.float32),
            jax.ShapeDtypeStruct((_NPAD,), jnp.int32),
        ),
        scratch_types=[
            pltpu.VMEM((9 * _CH,), jnp.float32),
            pltpu.VMEM((_CH,), jnp.int32),
            pltpu.VMEM((_CH,), jnp.float32),
            pltpu.VMEM((_CH,), jnp.int32),
            pltpu.VMEM((_CPAD,), jnp.int32),
            pltpu.VMEM((_CPAD,), jnp.float32),
            pltpu.VMEM((_CPAD,), jnp.float32),
        ],
    )(_k1_body)
    key_p, pre_p, rnk_p = k1(comp)

    k2 = functools.partial(
        pl.kernel, mesh=mesh, compiler_params=params,
        out_type=jax.ShapeDtypeStruct((_NW, 16), jnp.float32),
        scratch_types=[
            pltpu.VMEM((9 * _CH,), jnp.float32),
            pltpu.VMEM((_PUB,), jnp.int32),
            pltpu.VMEM((_PUB,), jnp.float32),
            pltpu.VMEM((_CH,), jnp.int32),
            pltpu.VMEM((16,), jnp.float32),
        ],
    )(_k2_body)
    return k2(comp, key_p, pre_p, rnk_p)


def kernel(centerness_flatten, centerness_targets, box_regression_flatten, reg_targets_flatten):
    comp = jnp.zeros((9, _NPAD), jnp.float32)
    comp = comp.at[0:4, :_N].set(reg_targets_flatten.T)
    comp = comp.at[4:8, :_N].set(box_regression_flatten.T)
    comp = comp.at[8, :_N].set(centerness_flatten)
    comp = comp.reshape(9 * _NPAD)
    partials = _run(comp)
    return jnp.sum(partials) / jnp.float32(_N - 1)


# speedup vs baseline: 2.6108x; 1.1360x over previous
import functools

import jax
import jax.numpy as jnp
from jax import lax
from jax.experimental import pallas as pl
from jax.experimental.pallas import tpu as pltpu
from jax.experimental.pallas import tpu_sc as plsc

_N = 5000
_NW = 32
_CH = 160
_NPAD = _NW * _CH
_CV = _CH // 16
_CPAD = 256
_NUM_PAD = _NPAD - _N
_IMAX = 2147483647
_GD = lax.GatherDimensionNumbers(
    offset_dims=(), collapsed_slice_dims=(0,), start_index_map=(0,)
)


def _perm(v, idx):
    return lax.gather(
        v, idx[:, None], _GD, slice_sizes=(1,),
        mode=lax.GatherScatterMode.PROMISE_IN_BOUNDS,
    )


def _iou_vecs(own_v, vj, iota, base):
    def comp(i):
        return own_v[pl.ds(i * _CH + vj * 16, 16)]
    p_l = comp(0)
    p_t = comp(1)
    p_r = comp(2)
    p_b = comp(3)
    t_l = comp(4)
    t_t = comp(5)
    t_r = comp(6)
    t_b = comp(7)
    cen = comp(8)
    target_area = (t_l + t_r) * (t_t + t_b)
    pred_area = (p_l + p_r) * (p_t + p_b)
    w_int = jnp.minimum(p_l, t_l) + jnp.minimum(p_r, t_r)
    h_int = jnp.minimum(p_b, t_b) + jnp.minimum(p_t, t_t)
    area_int = w_int * h_int
    area_union = target_area + pred_area - area_int
    iou = (area_int + 1.0) / (area_union + 1.0)
    gidx = iota + (base + vj * 16)
    valid = gidx < _N
    ikey = plsc.bitcast(jnp.where(valid, iou, 0.0), jnp.int32)
    return ikey, valid, cen


def _copy_own_rows(comp_hbm, own_v, base):
    for i in range(9):
        pltpu.sync_copy(
            comp_hbm.at[pl.ds(i * _NPAD + base, _CH)],
            own_v.at[pl.ds(i * _CH, _CH)],
        )


def _k1_body(comp_hbm, key_hbm, pre_hbm, rnk_hbm,
             own_v, ikey_v, b_v, rank_v, skey_v, sb_v, spre_v):
    cid = lax.axis_index("c")
    sid = lax.axis_index("s")
    wid = sid * 2 + cid
    base = wid * _CH
    iota = lax.iota(jnp.int32, 16)
    lane15 = jnp.full((16,), 15, jnp.int32)

    _copy_own_rows(comp_hbm, own_v, base)

    for vj in range(_CV):
        sl = pl.ds(vj * 16, 16)
        ikey, valid, cen = _iou_vecs(own_v, vj, iota, base)
        ikey_v[sl] = ikey
        b_v[sl] = jnp.where(valid, jnp.exp(-cen), 0.0)

    for vj in range(_CV, _CPAD // 16):
        sl = pl.ds(vj * 16, 16)
        skey_v[sl] = jnp.full((16,), _IMAX, jnp.int32)

    rots = [(iota + r) & 15 for r in range(16)]

    def rank_rv(rv, _):
        slr = pl.ds(rv * 16, 16)
        kr = ikey_v[slr]

        def rank_cv(cv, n_acc):
            kc = ikey_v[pl.ds(cv * 16, 16)]
            kadj = kr + jnp.where(jnp.full((16,), cv, jnp.int32) > rv, 1, 0)
            for r in range(16):
                kx = _perm(kc, rots[r])
                n_acc = n_acc + jnp.where(kx < kadj, 1, 0)
            return n_acc

        rank = lax.fori_loop(0, _CV, rank_cv, jnp.zeros((16,), jnp.int32))
        for r in range(1, 16):
            kx = _perm(kr, rots[r])
            rank = rank + jnp.where((kx == kr) & (rots[r] > iota), 1, 0)
        rank_v[slr] = rank
        plsc.store_scatter(skey_v, [rank], kr)
        plsc.store_scatter(sb_v, [rank], b_v[slr])
        return 0

    lax.fori_loop(0, _CV, rank_rv, 0)

    carry = jnp.zeros((16,), jnp.float32)
    for vj in range(_CV):
        sl = pl.ds(vj * 16, 16)
        bv = sb_v[sl]
        inc = plsc.cumsum(bv)
        spre_v[sl] = carry + (inc - bv)
        carry = carry + _perm(inc, lane15)
    spre_v[pl.ds(_CH, 16)] = carry

    pltpu.sync_copy(skey_v, key_hbm.at[pl.ds(wid * _CPAD, _CPAD)])
    pltpu.sync_copy(spre_v, pre_hbm.at[pl.ds(wid * _CPAD, _CPAD)])
    pltpu.sync_copy(rank_v, rnk_hbm.at[pl.ds(wid * _CH, _CH)])


def _k2_body(comp_hbm, key_hbm, pre_hbm, rnk_hbm, out_hbm,
             own_v, keyf_v, pref_v, rank_v, stage_v):
    cid = lax.axis_index("c")
    sid = lax.axis_index("s")
    wid = sid * 2 + cid
    base = wid * _CH
    iota = lax.iota(jnp.int32, 16)

    pltpu.sync_copy(key_hbm, keyf_v)
    pltpu.sync_copy(pre_hbm, pref_v)
    pltpu.sync_copy(rnk_hbm.at[pl.ds(base, _CH)], rank_v)
    _copy_own_rows(comp_hbm, own_v, base)

    def row_vreg(rv, acc):
        ikey, valid, cen = _iou_vecs(own_v, rv, iota, base)
        av = jnp.exp(-3.0 * cen)

        pos_own = rank_v[pl.ds(rv * 16, 16)]
        t0 = plsc.load_gather(pref_v, [pos_own + wid * _CPAD])

        def group(g, carry):
            t_acc, n_acc = carry
            cbase = g * 16
            pos = [jnp.zeros((16,), jnp.int32) for _ in range(16)]
            kadj = [
                ikey + jnp.where(
                    jnp.full((16,), cbase + cc, jnp.int32) > wid, 1, 0)
                for cc in range(16)
            ]
            for step in (128, 64, 32, 16, 8, 4, 2, 1):
                for cc in range(16):
                    c = cbase + cc
                    pk = plsc.load_gather(
                        keyf_v, [pos[cc] + (step - 1 + c * _CPAD)])
                    pos[cc] = pos[cc] + jnp.where(pk < kadj[cc], step, 0)
            for cc in range(16):
                c = cbase + cc
                skip = jnp.full((16,), c, jnp.int32) == wid
                p = jnp.where(skip, 0, pos[cc])
                n_acc = n_acc + p
                t_acc = t_acc + plsc.load_gather(pref_v, [p + c * _CPAD])
            return (t_acc, n_acc)

        t_vec, n_vec = lax.fori_loop(
            0, 2, group, (t0, pos_own))
        cnt = n_vec - _NUM_PAD
        ok_i = jnp.where(cnt > 0, 1, 0) * jnp.where(valid, 1, 0)
        cntf = jnp.where(cnt > 0, cnt, 1).astype(jnp.float32)
        return acc + jnp.where(ok_i > 0, av * t_vec / cntf, 0.0)

    acc = lax.fori_loop(0, _CV, row_vreg, jnp.zeros((16,), jnp.float32))
    stage_v[...] = acc
    pltpu.sync_copy(stage_v, out_hbm.at[wid])


_PUB = _NW * _CPAD


@jax.jit
def _run(comp):
    mesh = plsc.VectorSubcoreMesh(core_axis_name="c", subcore_axis_name="s")
    params = pltpu.CompilerParams(needs_layout_passes=False)

    k1 = functools.partial(
        pl.kernel, mesh=mesh, compiler_params=params,
        out_type=(
            jax.ShapeDtypeStruct((_PUB,), jnp.int32),
            jax.ShapeDtypeStruct((_PUB,), jnp.float32),
            jax.ShapeDtypeStruct((_NPAD,), jnp.int32),
        ),
        scratch_types=[
            pltpu.VMEM((9 * _CH,), jnp.float32),
            pltpu.VMEM((_CH,), jnp.int32),
            pltpu.VMEM((_CH,), jnp.float32),
            pltpu.VMEM((_CH,), jnp.int32),
            pltpu.VMEM((_CPAD,), jnp.int32),
            pltpu.VMEM((_CPAD,), jnp.float32),
            pltpu.VMEM((_CPAD,), jnp.float32),
        ],
    )(_k1_body)
    key_p, pre_p, rnk_p = k1(comp)

    k2 = functools.partial(
        pl.kernel, mesh=mesh, compiler_params=params,
        out_type=jax.ShapeDtypeStruct((_NW, 16), jnp.float32),
        scratch_types=[
            pltpu.VMEM((9 * _CH,), jnp.float32),
            pltpu.VMEM((_PUB,), jnp.int32),
            pltpu.VMEM((_PUB,), jnp.float32),
            pltpu.VMEM((_CH,), jnp.int32),
            pltpu.VMEM((16,), jnp.float32),
        ],
    )(_k2_body)
    return k2(comp, key_p, pre_p, rnk_p)


def kernel(centerness_flatten, centerness_targets, box_regression_flatten, reg_targets_flatten):
    comp = jnp.zeros((9, _NPAD), jnp.float32)
    comp = comp.at[0:4, :_N].set(reg_targets_flatten.T)
    comp = comp.at[4:8, :_N].set(box_regression_flatten.T)
    comp = comp.at[8, :_N].set(centerness_flatten)
    comp = comp.reshape(9 * _NPAD)
    partials = _run(comp)
    return jnp.sum(partials) / jnp.float32(_N - 1)


# speedup vs baseline: 3.0506x; 1.1684x over previous
import functools

import jax
import jax.numpy as jnp
from jax import lax
from jax.experimental import pallas as pl
from jax.experimental.pallas import tpu as pltpu
from jax.experimental.pallas import tpu_sc as plsc

_N = 5000
_NW = 32
_CH = 160
_NPAD = _NW * _CH
_CV = _CH // 16
_CPAD = 256
_NUM_PAD = _NPAD - _N
_IMAX = 2147483647
_GD = lax.GatherDimensionNumbers(
    offset_dims=(), collapsed_slice_dims=(0,), start_index_map=(0,)
)


def _perm(v, idx):
    return lax.gather(
        v, idx[:, None], _GD, slice_sizes=(1,),
        mode=lax.GatherScatterMode.PROMISE_IN_BOUNDS,
    )


def _iou_vecs(own_v, vj, iota, base):
    def comp(i):
        return own_v[pl.ds(i * _CH + vj * 16, 16)]
    p_l = comp(0)
    p_t = comp(1)
    p_r = comp(2)
    p_b = comp(3)
    t_l = comp(4)
    t_t = comp(5)
    t_r = comp(6)
    t_b = comp(7)
    cen = comp(8)
    target_area = (t_l + t_r) * (t_t + t_b)
    pred_area = (p_l + p_r) * (p_t + p_b)
    w_int = jnp.minimum(p_l, t_l) + jnp.minimum(p_r, t_r)
    h_int = jnp.minimum(p_b, t_b) + jnp.minimum(p_t, t_t)
    area_int = w_int * h_int
    area_union = target_area + pred_area - area_int
    iou = (area_int + 1.0) / (area_union + 1.0)
    gidx = iota + (base + vj * 16)
    valid = gidx < _N
    ikey = plsc.bitcast(jnp.where(valid, iou, 0.0), jnp.int32)
    return ikey, valid, cen


def _copy_own_rows(comp_hbm, own_v, base):
    for i in range(9):
        pltpu.sync_copy(
            comp_hbm.at[pl.ds(i * _NPAD + base, _CH)],
            own_v.at[pl.ds(i * _CH, _CH)],
        )


def _k1_body(comp_hbm, key_hbm, pre_hbm, rnk_hbm,
             own_v, ikey_v, b_v, rank_v, skey_v, sb_v, spre_v):
    cid = lax.axis_index("c")
    sid = lax.axis_index("s")
    wid = sid * 2 + cid
    base = wid * _CH
    iota = lax.iota(jnp.int32, 16)
    lane15 = jnp.full((16,), 15, jnp.int32)

    _copy_own_rows(comp_hbm, own_v, base)

    for vj in range(_CV):
        sl = pl.ds(vj * 16, 16)
        ikey, valid, cen = _iou_vecs(own_v, vj, iota, base)
        ikey_v[sl] = ikey
        b_v[sl] = jnp.where(valid, jnp.exp(-cen), 0.0)

    for vj in range(_CPAD // 16):
        skey_v[pl.ds(vj * 16, 16)] = jnp.full((16,), _IMAX, jnp.int32)

    rots = [(iota + r) & 15 for r in range(16)]

    def rank_rv(rv, _):
        slr = pl.ds(rv * 16, 16)
        kr = ikey_v[slr]

        def rank_cv(cv, n_acc):
            kc = ikey_v[pl.ds(cv * 16, 16)]
            kadj = kr + jnp.where(jnp.full((16,), cv, jnp.int32) > rv, 1, 0)
            for r in range(16):
                kx = _perm(kc, rots[r])
                n_acc = n_acc + jnp.where(kx < kadj, 1, 0)
            return n_acc

        rank = lax.fori_loop(0, _CV, rank_cv, jnp.zeros((16,), jnp.int32))
        for r in range(1, 16):
            kx = _perm(kr, rots[r])
            rank = rank + jnp.where((kx == kr) & (rots[r] > iota), 1, 0)
        rank_v[slr] = rank
        tr = ((rank & 15) << 4) | (rank >> 4)
        plsc.store_scatter(skey_v, [tr], kr)
        plsc.store_scatter(sb_v, [rank], b_v[slr])
        return 0

    lax.fori_loop(0, _CV, rank_rv, 0)

    carry = jnp.zeros((16,), jnp.float32)
    for vj in range(_CV):
        sl = pl.ds(vj * 16, 16)
        bv = sb_v[sl]
        inc = plsc.cumsum(bv)
        spre_v[sl] = carry + (inc - bv)
        carry = carry + _perm(inc, lane15)
    spre_v[pl.ds(_CH, 16)] = carry

    pltpu.sync_copy(skey_v, key_hbm.at[pl.ds(wid * _CPAD, _CPAD)])
    pltpu.sync_copy(spre_v, pre_hbm.at[pl.ds(wid * _CPAD, _CPAD)])
    pltpu.sync_copy(rank_v, rnk_hbm.at[pl.ds(wid * _CH, _CH)])


def _k2_body(comp_hbm, key_hbm, pre_hbm, rnk_hbm, out_hbm,
             own_v, keyf_v, pref_v, rank_v, stage_v):
    cid = lax.axis_index("c")
    sid = lax.axis_index("s")
    wid = sid * 2 + cid
    base = wid * _CH
    iota = lax.iota(jnp.int32, 16)

    pltpu.sync_copy(key_hbm, keyf_v)
    pltpu.sync_copy(pre_hbm, pref_v)
    pltpu.sync_copy(rnk_hbm.at[pl.ds(base, _CH)], rank_v)
    _copy_own_rows(comp_hbm, own_v, base)

    def row_vreg(rv, acc):
        ikey, valid, cen = _iou_vecs(own_v, rv, iota, base)
        av = jnp.exp(-3.0 * cen)

        pos_own = rank_v[pl.ds(rv * 16, 16)]
        t0 = plsc.load_gather(pref_v, [pos_own + wid * _CPAD])

        def group(g, carry):
            t_acc, n_acc = carry
            cbase = g * 16
            pos = [jnp.zeros((16,), jnp.int32) for _ in range(16)]
            kadj = [
                ikey + jnp.where(
                    jnp.full((16,), cbase + cc, jnp.int32) > wid, 1, 0)
                for cc in range(16)
            ]
            for step in (128, 64, 32, 16, 8, 4, 2, 1):
                for cc in range(16):
                    c = cbase + cc
                    p1 = pos[cc] + (step - 1)
                    pt = ((p1 & 15) << 4) | (p1 >> 4)
                    pk = plsc.load_gather(keyf_v, [pt + c * _CPAD])
                    pos[cc] = pos[cc] + jnp.where(pk < kadj[cc], step, 0)
            for cc in range(16):
                c = cbase + cc
                skip = jnp.full((16,), c, jnp.int32) == wid
                p = jnp.where(skip, 0, pos[cc])
                n_acc = n_acc + p
                t_acc = t_acc + plsc.load_gather(pref_v, [p + c * _CPAD])
            return (t_acc, n_acc)

        t_vec, n_vec = lax.fori_loop(
            0, 2, group, (t0, pos_own))
        cnt = n_vec - _NUM_PAD
        ok_i = jnp.where(cnt > 0, 1, 0) * jnp.where(valid, 1, 0)
        cntf = jnp.where(cnt > 0, cnt, 1).astype(jnp.float32)
        return acc + jnp.where(ok_i > 0, av * t_vec / cntf, 0.0)

    acc = lax.fori_loop(0, _CV, row_vreg, jnp.zeros((16,), jnp.float32))
    stage_v[...] = acc
    pltpu.sync_copy(stage_v, out_hbm.at[wid])


_PUB = _NW * _CPAD


@jax.jit
def _run(comp):
    mesh = plsc.VectorSubcoreMesh(core_axis_name="c", subcore_axis_name="s")
    params = pltpu.CompilerParams(needs_layout_passes=False)

    k1 = functools.partial(
        pl.kernel, mesh=mesh, compiler_params=params,
        out_type=(
            jax.ShapeDtypeStruct((_PUB,), jnp.int32),
            jax.ShapeDtypeStruct((_PUB,), jnp.float32),
            jax.ShapeDtypeStruct((_NPAD,), jnp.int32),
        ),
        scratch_types=[
            pltpu.VMEM((9 * _CH,), jnp.float32),
            pltpu.VMEM((_CH,), jnp.int32),
            pltpu.VMEM((_CH,), jnp.float32),
            pltpu.VMEM((_CH,), jnp.int32),
            pltpu.VMEM((_CPAD,), jnp.int32),
            pltpu.VMEM((_CPAD,), jnp.float32),
            pltpu.VMEM((_CPAD,), jnp.float32),
        ],
    )(_k1_body)
    key_p, pre_p, rnk_p = k1(comp)

    k2 = functools.partial(
        pl.kernel, mesh=mesh, compiler_params=params,
        out_type=jax.ShapeDtypeStruct((_NW, 16), jnp.float32),
        scratch_types=[
            pltpu.VMEM((9 * _CH,), jnp.float32),
            pltpu.VMEM((_PUB,), jnp.int32),
            pltpu.VMEM((_PUB,), jnp.float32),
            pltpu.VMEM((_CH,), jnp.int32),
            pltpu.VMEM((16,), jnp.float32),
        ],
    )(_k2_body)
    return k2(comp, key_p, pre_p, rnk_p)


def kernel(centerness_flatten, centerness_targets, box_regression_flatten, reg_targets_flatten):
    comp = jnp.zeros((9, _NPAD), jnp.float32)
    comp = comp.at[0:4, :_N].set(reg_targets_flatten.T)
    comp = comp.at[4:8, :_N].set(box_regression_flatten.T)
    comp = comp.at[8, :_N].set(centerness_flatten)
    comp = comp.reshape(9 * _NPAD)
    partials = _run(comp)
    return jnp.sum(partials) / jnp.float32(_N - 1)
